# Initial kernel scaffold; baseline (speedup 1.0000x reference)
import jax
import jax.numpy as jnp
from jax.experimental import pallas as pl


def kernel(cat_tensor, tables):
    raise NotImplementedError("write your pallas kernel here")


# speedup vs baseline: 3.2380x; 3.2380x over previous
import functools

import jax
import jax.numpy as jnp
from jax import lax
from jax.experimental import pallas as pl
from jax.experimental.pallas import tpu as pltpu
from jax.experimental.pallas import tpu_sc as plsc

N_COLS = 26
VOCAB = 100
DIM = 50
DIM_PAD = 56
BATCH = 16384

NC, NS, L = 2, 16, 16
NW = NC * NS
ROWS = BATCH * N_COLS
R_PER_W = ROWS // NW
CHUNK = 1024
N_CHUNKS = R_PER_W // CHUNK
GSZ = 128
N_G = CHUNK // GSZ


def _make_kernel():
    mesh = plsc.VectorSubcoreMesh(core_axis_name="c", subcore_axis_name="s")

    @functools.partial(
        pl.kernel,
        out_type=jax.ShapeDtypeStruct((ROWS, DIM_PAD), jnp.float32),
        mesh=mesh,
        scratch_types=[
            pltpu.VMEM((N_G, GSZ), jnp.int32),
            pltpu.VMEM((CHUNK, DIM_PAD), jnp.float32),
            pltpu.SemaphoreType.DMA,
        ],
        compiler_params=pltpu.CompilerParams(use_tc_tiling_on_sc=False),
    )
    def emb(idx_hbm, tab_hbm, out_hbm, fidx_v, rows_v, sem):
        w = lax.axis_index("s") * NC + lax.axis_index("c")
        lane = lax.iota(jnp.int32, L)

        def chunk_body(ci, carry):
            base = w * R_PER_W + ci * CHUNK
            row0 = base // GSZ
            pltpu.sync_copy(idx_hbm.at[pl.ds(row0, N_G)], fidx_v)

            def vec_body(s, carry2):
                g, s2 = s // (GSZ // L), s % (GSZ // L)
                r = base + s * L + lane
                col = lax.rem(r, N_COLS)
                fidx_v[g, pl.ds(s2 * L, L)] = (
                    fidx_v[g, pl.ds(s2 * L, L)] + col * VOCAB
                )
                return carry2

            lax.fori_loop(0, CHUNK // L, vec_body, 0)

            copies = [
                pltpu.async_copy(
                    tab_hbm.at[fidx_v.at[g]],
                    rows_v.at[pl.ds(g * GSZ, GSZ)],
                    sem,
                )
                for g in range(N_G)
            ]
            for c in copies:
                c.wait()
            pltpu.sync_copy(rows_v, out_hbm.at[pl.ds(base, CHUNK)])
            return carry

        lax.fori_loop(0, N_CHUNKS, chunk_body, 0)

    return emb


_emb = _make_kernel()


def kernel(cat_tensor, tables):
    idx_2d = cat_tensor.reshape(ROWS // GSZ, GSZ)
    tab_flat = tables.reshape(N_COLS * VOCAB, DIM)
    tab_pad = jnp.pad(tab_flat, ((0, 0), (0, DIM_PAD - DIM)))
    out = _emb(idx_2d, tab_pad)
    return out[:, :DIM].reshape(BATCH, N_COLS * DIM)


# speedup vs baseline: 6.2845x; 1.9409x over previous
import functools

import jax
import jax.numpy as jnp
from jax import lax
from jax.experimental import pallas as pl
from jax.experimental.pallas import tpu as pltpu
from jax.experimental.pallas import tpu_sc as plsc

N_COLS = 26
VOCAB = 100
DIM = 50
BATCH = 16384
OUTC = N_COLS * DIM

NC, NS, L = 2, 16, 16
NW = NC * NS
BBLK = 1024
NB = BATCH // BBLK
N_UNITS = N_COLS * NB
U_PER_W = N_UNITS // NW
NV = BBLK // L


def _make_kernel():
    mesh = plsc.VectorSubcoreMesh(core_axis_name="c", subcore_axis_name="s")

    @functools.partial(
        pl.kernel,
        out_type=jax.ShapeDtypeStruct((OUTC, BATCH), jnp.float32),
        mesh=mesh,
        scratch_types=[
            pltpu.VMEM((2, BBLK), jnp.int32),
            pltpu.VMEM((2, DIM * VOCAB), jnp.float32),
            pltpu.VMEM((2 * DIM, BBLK), jnp.float32),
            pltpu.SemaphoreType.DMA,
            pltpu.SemaphoreType.DMA,
        ],
        compiler_params=pltpu.CompilerParams(use_tc_tiling_on_sc=False, needs_layout_passes=False),
    )
    def emb(cat_t_hbm, tab_t_hbm, out_hbm, idx_v, tab_v, out_v, sem_in, sem_out):
        w = lax.axis_index("s") * NC + lax.axis_index("c")

        def load_unit(k, slot):
            u = w * U_PER_W + k
            i, tb = u // NB, u % NB
            cp_i = pltpu.async_copy(
                cat_t_hbm.at[i, pl.ds(tb * BBLK, BBLK)], idx_v.at[slot], sem_in
            )
            cp_t = pltpu.async_copy(tab_t_hbm.at[i], tab_v.at[slot], sem_in)
            return cp_i, cp_t

        def compute_unit(slot):
            def vec_body(s, carry):
                iv = idx_v[slot, pl.ds(s * L, L)]
                for j in range(DIM):
                    out_v[slot * DIM + j, pl.ds(s * L, L)] = plsc.load_gather(
                        tab_v.at[slot], [iv + j * VOCAB]
                    )
                return carry

            lax.fori_loop(0, NV, vec_body, 0)

        def store_unit(k, slot):
            u = w * U_PER_W + k
            i, tb = u // NB, u % NB
            return pltpu.async_copy(
                out_v.at[pl.ds(slot * DIM, DIM)],
                out_hbm.at[pl.ds(i * DIM, DIM), pl.ds(tb * BBLK, BBLK)],
                sem_out,
            )

        loads = load_unit(0, 0)
        stores = [None, None]
        for k in range(U_PER_W):
            slot = k % 2
            for cp in loads:
                cp.wait()
            if k + 1 < U_PER_W:
                loads = load_unit(k + 1, (k + 1) % 2)
            if stores[slot] is not None:
                stores[slot].wait()
            compute_unit(slot)
            stores[slot] = store_unit(k, slot)
        for st in stores:
            if st is not None:
                st.wait()

    return emb


_emb = _make_kernel()


def kernel(cat_tensor, tables):
    cat_t = cat_tensor.T
    tab_t = tables.transpose(0, 2, 1).reshape(N_COLS, DIM * VOCAB)
    out_t = _emb(cat_t, tab_t)
    return out_t.T


# speedup vs baseline: 11.3646x; 1.8083x over previous
import functools

import jax
import jax.numpy as jnp
from jax import lax
from jax.experimental import pallas as pl
from jax.experimental.pallas import tpu as pltpu
from jax.experimental.pallas import tpu_sc as plsc

N_COLS = 26
VOCAB = 100
DIM = 50
BATCH = 16384
OUTC = N_COLS * DIM

NC, NS, L = 2, 16, 16
NW = NC * NS
BBLK = 1024
NB = BATCH // BBLK
N_UNITS = N_COLS * NB
U_PER_W = N_UNITS // NW
NV = BBLK // L


def _make_kernel():
    mesh = plsc.VectorSubcoreMesh(core_axis_name="c", subcore_axis_name="s")

    @functools.partial(
        pl.kernel,
        out_type=jax.ShapeDtypeStruct((OUTC, BATCH), jnp.float32),
        mesh=mesh,
        scratch_types=[
            pltpu.VMEM((2, BBLK), jnp.int32),
            pltpu.VMEM((2, DIM * VOCAB), jnp.float32),
            pltpu.VMEM((2 * DIM, BBLK), jnp.float32),
            pltpu.SemaphoreType.DMA,
            pltpu.SemaphoreType.DMA,
        ],
        compiler_params=pltpu.CompilerParams(use_tc_tiling_on_sc=False, needs_layout_passes=False),
    )
    def emb(cat_t_hbm, tab_t_hbm, out_hbm, idx_v, tab_v, out_v, sem_in, sem_out):
        w = lax.axis_index("s") * NC + lax.axis_index("c")

        def load_unit(k, slot):
            u = w * U_PER_W + k
            i, tb = u // NB, u % NB
            cp_i = pltpu.async_copy(
                cat_t_hbm.at[i, pl.ds(tb * BBLK, BBLK)], idx_v.at[slot], sem_in
            )
            cp_t = pltpu.async_copy(tab_t_hbm.at[i], tab_v.at[slot], sem_in)
            return cp_i, cp_t

        def compute_unit(slot):
            def vec_body(s, carry):
                iv = idx_v[slot, pl.ds(s * L, L)]
                for j0 in range(0, DIM, 10):
                    vals = [
                        plsc.load_gather(tab_v.at[slot], [iv + j * VOCAB])
                        for j in range(j0, j0 + 10)
                    ]
                    for j, v in zip(range(j0, j0 + 10), vals):
                        out_v[slot * DIM + j, pl.ds(s * L, L)] = v
                return carry

            lax.fori_loop(0, NV, vec_body, 0)

        def store_unit(k, slot):
            u = w * U_PER_W + k
            i, tb = u // NB, u % NB
            return pltpu.async_copy(
                out_v.at[pl.ds(slot * DIM, DIM)],
                out_hbm.at[pl.ds(i * DIM, DIM), pl.ds(tb * BBLK, BBLK)],
                sem_out,
            )

        loads = load_unit(0, 0)
        stores = [None, None]
        for k in range(U_PER_W):
            slot = k % 2
            for cp in loads:
                cp.wait()
            if k + 1 < U_PER_W:
                loads = load_unit(k + 1, (k + 1) % 2)
            if stores[slot] is not None:
                stores[slot].wait()
            compute_unit(slot)
            stores[slot] = store_unit(k, slot)
        for st in stores:
            if st is not None:
                st.wait()

    return emb


_emb = _make_kernel()


def kernel(cat_tensor, tables):
    cat_t = cat_tensor.T
    tab_t = tables.transpose(0, 2, 1).reshape(N_COLS, DIM * VOCAB)
    out_t = _emb(cat_t, tab_t)
    return out_t.T


# speedup vs baseline: 19.0947x; 1.6802x over previous
import functools

import jax
import jax.numpy as jnp
from jax import lax
from jax.experimental import pallas as pl
from jax.experimental.pallas import tpu as pltpu
from jax.experimental.pallas import tpu_sc as plsc

N_COLS = 26
VOCAB = 100
DIM = 50
BATCH = 16384
OUTC = N_COLS * DIM
OUTC_PAD = 1304
NTR = OUTC_PAD // 8

NC, NS, L = 2, 16, 16
NW = NC * NS
BBLK = BATCH // NW
NT = BBLK // 128
NV = BBLK // L
GB = 10


def _make_kernel():
    mesh = plsc.VectorSubcoreMesh(core_axis_name="c", subcore_axis_name="s")

    @functools.partial(
        pl.kernel,
        out_type=jax.ShapeDtypeStruct((NTR, BATCH // 128, 8, 128), jnp.float32),
        mesh=mesh,
        scratch_types=[
            pltpu.VMEM((2, BBLK), jnp.int32),
            pltpu.VMEM((2, DIM * VOCAB), jnp.float32),
            pltpu.VMEM((16, NT, 8, 128), jnp.float32),
            pltpu.SemaphoreType.DMA,
            pltpu.SemaphoreType.DMA,
        ],
        compiler_params=pltpu.CompilerParams(
            use_tc_tiling_on_sc=False, needs_layout_passes=False
        ),
    )
    def emb(cat_t_hbm, tab_t_hbm, out_hbm, idx_v, tab_v, out_v, sem_in, sem_out):
        w = lax.axis_index("s") * NC + lax.axis_index("c")
        b0 = w * BBLK
        t0 = w * NT

        def load_unit(i, slot):
            cp_i = pltpu.async_copy(
                cat_t_hbm.at[i, pl.ds(b0, BBLK)], idx_v.at[slot], sem_in
            )
            cp_t = pltpu.async_copy(tab_t_hbm.at[i], tab_v.at[slot], sem_in)
            return cp_i, cp_t

        def compute_unit(i, slot):
            phi = (DIM * i) % 8

            def vec_body(s, carry):
                t, sl = s // 8, s % 8
                iv = idx_v[slot, pl.ds(s * L, L)]
                for j0 in range(0, DIM, GB):
                    vals = [
                        plsc.load_gather(tab_v.at[slot], [iv + jl * VOCAB])
                        for jl in range(j0, j0 + GB)
                    ]
                    for jl, v in zip(range(j0, j0 + GB), vals):
                        al, sub = (phi + jl) // 8, (phi + jl) % 8
                        out_v[slot * 8 + al, t, sub, pl.ds(sl * L, L)] = v
                return carry

            lax.fori_loop(0, NV, vec_body, 0)

        def store_unit(i, slot):
            phi = (DIM * i) % 8
            a0 = (DIM * i) // 8
            span = (phi + DIM + 7) // 8
            copies = []
            for al in range(span):
                r0 = phi if al == 0 else 0
                r1 = phi + DIM - 8 * al
                r1 = 8 if r1 > 8 else r1
                rn = r1 - r0
                copies.append(
                    pltpu.async_copy(
                        out_v.at[slot * 8 + al, :, pl.ds(r0, rn), :],
                        out_hbm.at[a0 + al, pl.ds(t0, NT), pl.ds(r0, rn), :],
                        sem_out,
                    )
                )
            return copies

        loads = load_unit(0, 0)
        stores = [None, None]
        for i in range(N_COLS):
            slot = i % 2
            for cp in loads:
                cp.wait()
            if i + 1 < N_COLS:
                loads = load_unit(i + 1, (i + 1) % 2)
            if stores[slot] is not None:
                for cp in stores[slot]:
                    cp.wait()
                stores[slot] = None
            compute_unit(i, slot)
            stores[slot] = store_unit(i, slot)
        for st in stores:
            if st is not None:
                for cp in st:
                    cp.wait()

    return emb


_emb = _make_kernel()


def kernel(cat_tensor, tables):
    cat_t = cat_tensor.T
    tab_t = tables.transpose(0, 2, 1).reshape(N_COLS, DIM * VOCAB)
    out4 = _emb(cat_t, tab_t)
    out = out4.transpose(0, 2, 1, 3).reshape(OUTC_PAD, BATCH).T
    return out[:, :OUTC]
